# baseline (speedup 1.0000x reference)
import functools

import jax
import jax.numpy as jnp
from jax import lax
from jax.experimental import pallas as pl
from jax.experimental.pallas import tpu as pltpu

D_MODEL = 1024
HIDDEN = 2048
N_EXPERTS = 8
TOP_K = 2

TM = 512

_INTERPRET = False


def _grouped_swiglu_kernel(meta_ref, x_ref, wg_ref, wu_ref, wd_ref, o_ref):
    i = pl.program_id(0)
    xb = x_ref[...]
    a = jnp.dot(xb, wg_ref[0], preferred_element_type=jnp.float32)
    b = jnp.dot(xb, wu_ref[0], preferred_element_type=jnp.float32)
    g = ((a * jax.nn.sigmoid(a)) * b).astype(jnp.bfloat16)
    contrib = jnp.dot(g, wd_ref[0], preferred_element_type=jnp.float32)
    rem = meta_ref[1, i]
    rows = lax.broadcasted_iota(jnp.int32, (TM, 1), 0)
    o_ref[...] = jnp.where(rows < rem, contrib, 0.0).astype(jnp.bfloat16)


def _grouped_swiglu(meta, xs, Wg, Wu, Wd, nt):
    np_rows = nt * TM
    grid_spec = pltpu.PrefetchScalarGridSpec(
        num_scalar_prefetch=1,
        grid=(nt,),
        in_specs=[
            pl.BlockSpec((TM, D_MODEL), lambda i, m: (i, 0)),
            pl.BlockSpec((1, D_MODEL, HIDDEN), lambda i, m: (m[0, i], 0, 0)),
            pl.BlockSpec((1, D_MODEL, HIDDEN), lambda i, m: (m[0, i], 0, 0)),
            pl.BlockSpec((1, HIDDEN, D_MODEL), lambda i, m: (m[0, i], 0, 0)),
        ],
        out_specs=pl.BlockSpec((TM, D_MODEL), lambda i, m: (i, 0)),
    )
    return pl.pallas_call(
        _grouped_swiglu_kernel,
        grid_spec=grid_spec,
        out_shape=jax.ShapeDtypeStruct((np_rows, D_MODEL), jnp.bfloat16),
        compiler_params=pltpu.CompilerParams(
            dimension_semantics=("arbitrary",)),
        interpret=_INTERPRET,
    )(meta, xs, Wg, Wu, Wd)


def _shared_swiglu_kernel(x_ref, wg_ref, wu_ref, wd_ref, o_ref):
    xb = x_ref[...]
    a = jnp.dot(xb, wg_ref[...], preferred_element_type=jnp.float32)
    b = jnp.dot(xb, wu_ref[...], preferred_element_type=jnp.float32)
    g = ((a * jax.nn.sigmoid(a)) * b).astype(jnp.bfloat16)
    o_ref[...] = jnp.dot(g, wd_ref[...], preferred_element_type=jnp.float32)


def _shared_swiglu(xb16, sWg, sWu, sWd):
    t = xb16.shape[0]
    return pl.pallas_call(
        _shared_swiglu_kernel,
        grid=(t // TM,),
        in_specs=[
            pl.BlockSpec((TM, D_MODEL), lambda i: (i, 0)),
            pl.BlockSpec((D_MODEL, HIDDEN), lambda i: (0, 0)),
            pl.BlockSpec((D_MODEL, HIDDEN), lambda i: (0, 0)),
            pl.BlockSpec((HIDDEN, D_MODEL), lambda i: (0, 0)),
        ],
        out_specs=pl.BlockSpec((TM, D_MODEL), lambda i: (i, 0)),
        out_shape=jax.ShapeDtypeStruct((t, D_MODEL), jnp.float32),
        compiler_params=pltpu.CompilerParams(
            dimension_semantics=("arbitrary",)),
        interpret=_INTERPRET,
    )(xb16, sWg, sWu, sWd)


def _combine_kernel(sh_ref, b0_ref, b1_ref, g0_ref, g1_ref, o_ref):
    o_ref[...] = (sh_ref[...]
                  + g0_ref[:, :1] * b0_ref[...].astype(jnp.float32)
                  + g1_ref[:, :1] * b1_ref[...].astype(jnp.float32))


def _combine(shared, buf, g0, g1):
    t = shared.shape[0]
    nb = t // TM
    return pl.pallas_call(
        _combine_kernel,
        grid=(nb,),
        in_specs=[
            pl.BlockSpec((TM, D_MODEL), lambda i: (i, 0)),
            pl.BlockSpec((TM, D_MODEL), lambda i: (i, 0)),
            pl.BlockSpec((TM, D_MODEL), lambda i, nb=nb: (i + nb, 0)),
            pl.BlockSpec((TM, 128), lambda i: (i, 0)),
            pl.BlockSpec((TM, 128), lambda i: (i, 0)),
        ],
        out_specs=pl.BlockSpec((TM, D_MODEL), lambda i: (i, 0)),
        out_shape=jax.ShapeDtypeStruct((t, D_MODEL), jnp.float32),
        compiler_params=pltpu.CompilerParams(
            dimension_semantics=("arbitrary",)),
        interpret=_INTERPRET,
    )(shared, buf, buf, g0, g1)


def kernel(x, Wr, Wg, Wu, Wd, sWg, sWu, sWd):
    b, s, d = x.shape
    t = b * s
    a_total = t * TOP_K
    nt = a_total // TM + N_EXPERTS
    np_rows = nt * TM
    x_flat = x.reshape(t, d)
    xb16 = x_flat.astype(jnp.bfloat16)
    Wg = Wg.astype(jnp.bfloat16)
    Wu = Wu.astype(jnp.bfloat16)
    Wd = Wd.astype(jnp.bfloat16)
    sWg = sWg.astype(jnp.bfloat16)
    sWu = sWu.astype(jnp.bfloat16)
    sWd = sWd.astype(jnp.bfloat16)

    logits = x_flat @ Wr
    idx1 = jnp.argmax(logits, axis=-1)
    l1 = jnp.max(logits, axis=-1)
    masked = jnp.where(jnp.arange(N_EXPERTS)[None, :] == idx1[:, None],
                       -jnp.inf, logits)
    idx2 = jnp.argmax(masked, axis=-1)
    l2 = jnp.max(masked, axis=-1)
    m = jnp.maximum(l1, l2)
    e1 = jnp.exp(l1 - m)
    e2 = jnp.exp(l2 - m)
    zs = e1 + e2
    gate = jnp.stack([e1 / zs, e2 / zs], axis=-1)
    top_idx = jnp.stack([idx1, idx2], axis=-1).astype(jnp.int32)

    e_flat = jnp.concatenate([top_idx[:, 0], top_idx[:, 1]])
    onehot = (e_flat[:, None] == jnp.arange(N_EXPERTS)[None, :]).astype(jnp.int32)
    counts = onehot.sum(axis=0)
    nt_e = (counts + TM - 1) // TM
    cum_nt = jnp.cumsum(nt_e)
    first_tile = cum_nt - nt_e
    nt_total = cum_nt[-1]
    group_start = first_tile * TM
    rank = jnp.sum((jnp.cumsum(onehot, axis=0) - onehot) * onehot, axis=1)
    pos = group_start[e_flat] + rank
    tok = jnp.concatenate([jnp.arange(t, dtype=jnp.int32)] * 2)
    sorted_tok = jnp.zeros((np_rows,), jnp.int32).at[pos].set(tok)

    ti = jnp.arange(nt, dtype=jnp.int32)
    tile_e = jnp.clip(jnp.searchsorted(cum_nt, ti, side='right'), 0,
                      N_EXPERTS - 1).astype(jnp.int32)
    rem = counts[tile_e] - (ti - first_tile[tile_e]) * TM
    rem = jnp.where(ti < nt_total, rem, 0)
    meta = jnp.stack([tile_e, rem]).astype(jnp.int32)

    x_i32 = lax.bitcast_convert_type(
        xb16.reshape(t, d // 2, 2), jnp.int32)
    xs = lax.bitcast_convert_type(
        jnp.take(x_i32, sorted_tok, axis=0), jnp.bfloat16
    ).reshape(np_rows, d)

    shared = _shared_swiglu(xb16, sWg, sWu, sWd)

    y = _grouped_swiglu(meta, xs, Wg, Wu, Wd, nt)

    y_i32 = lax.bitcast_convert_type(
        y.reshape(np_rows, d // 2, 2), jnp.int32)
    buf = lax.bitcast_convert_type(
        jnp.take(y_i32, pos, axis=0), jnp.bfloat16
    ).reshape(a_total, d)
    g0 = jnp.broadcast_to(gate[:, 0:1], (t, 128))
    g1 = jnp.broadcast_to(gate[:, 1:2], (t, 128))

    out = _combine(shared, buf, g0, g1)
    return out.reshape(b, s, d)


# speedup vs baseline: 2.2284x; 2.2284x over previous
import functools

import jax
import jax.numpy as jnp
from jax import lax
from jax.experimental import pallas as pl
from jax.experimental.pallas import tpu as pltpu

D_MODEL = 1024
HIDDEN = 2048
N_EXPERTS = 8
TOP_K = 2

TM = 512

_INTERPRET = False


def _grouped_swiglu_kernel(meta_ref, x_ref, wg_ref, wu_ref, wd_ref, o_ref):
    i = pl.program_id(0)
    xb = x_ref[...].astype(jnp.bfloat16)
    a = jnp.dot(xb, wg_ref[0], preferred_element_type=jnp.float32)
    b = jnp.dot(xb, wu_ref[0], preferred_element_type=jnp.float32)
    g = ((a * jax.nn.sigmoid(a)) * b).astype(jnp.bfloat16)
    contrib = jnp.dot(g, wd_ref[0], preferred_element_type=jnp.float32)
    rem = meta_ref[1, i]
    rows = lax.broadcasted_iota(jnp.int32, (TM, 1), 0)
    o_ref[...] = jnp.where(rows < rem, contrib, 0.0)


def _grouped_swiglu(meta, xs, Wg, Wu, Wd, nt):
    np_rows = nt * TM
    grid_spec = pltpu.PrefetchScalarGridSpec(
        num_scalar_prefetch=1,
        grid=(nt,),
        in_specs=[
            pl.BlockSpec((TM, D_MODEL), lambda i, m: (i, 0)),
            pl.BlockSpec((1, D_MODEL, HIDDEN), lambda i, m: (m[0, i], 0, 0)),
            pl.BlockSpec((1, D_MODEL, HIDDEN), lambda i, m: (m[0, i], 0, 0)),
            pl.BlockSpec((1, HIDDEN, D_MODEL), lambda i, m: (m[0, i], 0, 0)),
        ],
        out_specs=pl.BlockSpec((TM, D_MODEL), lambda i, m: (i, 0)),
    )
    return pl.pallas_call(
        _grouped_swiglu_kernel,
        grid_spec=grid_spec,
        out_shape=jax.ShapeDtypeStruct((np_rows, D_MODEL), jnp.float32),
        compiler_params=pltpu.CompilerParams(
            dimension_semantics=("arbitrary",)),
        interpret=_INTERPRET,
    )(meta, xs, Wg, Wu, Wd)


def _shared_swiglu_kernel(x_ref, wg_ref, wu_ref, wd_ref, o_ref):
    xb = x_ref[...].astype(jnp.bfloat16)
    a = jnp.dot(xb, wg_ref[...], preferred_element_type=jnp.float32)
    b = jnp.dot(xb, wu_ref[...], preferred_element_type=jnp.float32)
    g = ((a * jax.nn.sigmoid(a)) * b).astype(jnp.bfloat16)
    o_ref[...] = jnp.dot(g, wd_ref[...], preferred_element_type=jnp.float32)


def _shared_swiglu(x_flat, sWg, sWu, sWd):
    t = x_flat.shape[0]
    return pl.pallas_call(
        _shared_swiglu_kernel,
        grid=(t // TM,),
        in_specs=[
            pl.BlockSpec((TM, D_MODEL), lambda i: (i, 0)),
            pl.BlockSpec((D_MODEL, HIDDEN), lambda i: (0, 0)),
            pl.BlockSpec((D_MODEL, HIDDEN), lambda i: (0, 0)),
            pl.BlockSpec((HIDDEN, D_MODEL), lambda i: (0, 0)),
        ],
        out_specs=pl.BlockSpec((TM, D_MODEL), lambda i: (i, 0)),
        out_shape=jax.ShapeDtypeStruct((t, D_MODEL), jnp.float32),
        compiler_params=pltpu.CompilerParams(
            dimension_semantics=("arbitrary",)),
        interpret=_INTERPRET,
    )(x_flat, sWg, sWu, sWd)


def _combine_kernel(sh_ref, b0_ref, b1_ref, g0_ref, g1_ref, o_ref):
    o_ref[...] = (sh_ref[...]
                  + g0_ref[:, :1] * b0_ref[...]
                  + g1_ref[:, :1] * b1_ref[...])


def _combine(shared, buf, g0, g1):
    t = shared.shape[0]
    nb = t // TM
    return pl.pallas_call(
        _combine_kernel,
        grid=(nb,),
        in_specs=[
            pl.BlockSpec((TM, D_MODEL), lambda i: (i, 0)),
            pl.BlockSpec((TM, D_MODEL), lambda i: (i, 0)),
            pl.BlockSpec((TM, D_MODEL), lambda i, nb=nb: (i + nb, 0)),
            pl.BlockSpec((TM, 128), lambda i: (i, 0)),
            pl.BlockSpec((TM, 128), lambda i: (i, 0)),
        ],
        out_specs=pl.BlockSpec((TM, D_MODEL), lambda i: (i, 0)),
        out_shape=jax.ShapeDtypeStruct((t, D_MODEL), jnp.float32),
        compiler_params=pltpu.CompilerParams(
            dimension_semantics=("arbitrary",)),
        interpret=_INTERPRET,
    )(shared, buf, buf, g0, g1)


def kernel(x, Wr, Wg, Wu, Wd, sWg, sWu, sWd):
    b, s, d = x.shape
    t = b * s
    a_total = t * TOP_K
    nt = a_total // TM + N_EXPERTS
    np_rows = nt * TM
    x_flat = x.reshape(t, d)
    Wg = Wg.astype(jnp.bfloat16)
    Wu = Wu.astype(jnp.bfloat16)
    Wd = Wd.astype(jnp.bfloat16)
    sWg = sWg.astype(jnp.bfloat16)
    sWu = sWu.astype(jnp.bfloat16)
    sWd = sWd.astype(jnp.bfloat16)

    logits = x_flat @ Wr
    idx1 = jnp.argmax(logits, axis=-1)
    l1 = jnp.max(logits, axis=-1)
    masked = jnp.where(jnp.arange(N_EXPERTS)[None, :] == idx1[:, None],
                       -jnp.inf, logits)
    idx2 = jnp.argmax(masked, axis=-1)
    l2 = jnp.max(masked, axis=-1)
    m = jnp.maximum(l1, l2)
    e1 = jnp.exp(l1 - m)
    e2 = jnp.exp(l2 - m)
    zs = e1 + e2
    gate = jnp.stack([e1 / zs, e2 / zs], axis=-1)
    top_idx = jnp.stack([idx1, idx2], axis=-1).astype(jnp.int32)

    e_flat = jnp.concatenate([top_idx[:, 0], top_idx[:, 1]])
    onehot = (e_flat[:, None] == jnp.arange(N_EXPERTS)[None, :]).astype(jnp.int32)
    counts = onehot.sum(axis=0)
    nt_e = (counts + TM - 1) // TM
    cum_nt = jnp.cumsum(nt_e)
    first_tile = cum_nt - nt_e
    nt_total = cum_nt[-1]
    group_start = first_tile * TM
    rank = jnp.sum((jnp.cumsum(onehot, axis=0) - onehot) * onehot, axis=1)
    pos = group_start[e_flat] + rank
    tok = jnp.concatenate([jnp.arange(t, dtype=jnp.int32)] * 2)
    sorted_tok = jnp.zeros((np_rows,), jnp.int32).at[pos].set(tok)

    ti = jnp.arange(nt, dtype=jnp.int32)
    tile_e = jnp.clip(jnp.searchsorted(cum_nt, ti, side='right'), 0,
                      N_EXPERTS - 1).astype(jnp.int32)
    rem = counts[tile_e] - (ti - first_tile[tile_e]) * TM
    rem = jnp.where(ti < nt_total, rem, 0)
    meta = jnp.stack([tile_e, rem]).astype(jnp.int32)

    xs = jnp.take(x_flat, sorted_tok, axis=0)

    shared = _shared_swiglu(x_flat, sWg, sWu, sWd)

    y = _grouped_swiglu(meta, xs, Wg, Wu, Wd, nt)

    buf = jnp.take(y, pos, axis=0)
    g0 = jnp.broadcast_to(gate[:, 0:1], (t, 128))
    g1 = jnp.broadcast_to(gate[:, 1:2], (t, 128))

    out = _combine(shared, buf, g0, g1)
    return out.reshape(b, s, d)


# speedup vs baseline: 2.2333x; 1.0022x over previous
import functools

import jax
import jax.numpy as jnp
from jax import lax
from jax.experimental import pallas as pl
from jax.experimental.pallas import tpu as pltpu

D_MODEL = 1024
HIDDEN = 2048
N_EXPERTS = 8
TOP_K = 2

TM = 512

_INTERPRET = False


def _grouped_swiglu_kernel(meta_ref, x_ref, wg_ref, wu_ref, wd_ref, o_ref):
    i = pl.program_id(0)
    xb = x_ref[...].astype(jnp.bfloat16)
    a = jnp.dot(xb, wg_ref[0], preferred_element_type=jnp.float32)
    b = jnp.dot(xb, wu_ref[0], preferred_element_type=jnp.float32)
    g = ((a * jax.nn.sigmoid(a)) * b).astype(jnp.bfloat16)
    contrib = jnp.dot(g, wd_ref[0], preferred_element_type=jnp.float32)
    rem = meta_ref[1, i]
    rows = lax.broadcasted_iota(jnp.int32, (TM, 1), 0)
    o_ref[...] = jnp.where(rows < rem, contrib, 0.0)


def _grouped_swiglu(meta, xs, Wg, Wu, Wd, nt):
    np_rows = nt * TM
    grid_spec = pltpu.PrefetchScalarGridSpec(
        num_scalar_prefetch=1,
        grid=(nt,),
        in_specs=[
            pl.BlockSpec((TM, D_MODEL), lambda i, m: (i, 0)),
            pl.BlockSpec((1, D_MODEL, HIDDEN), lambda i, m: (m[0, i], 0, 0)),
            pl.BlockSpec((1, D_MODEL, HIDDEN), lambda i, m: (m[0, i], 0, 0)),
            pl.BlockSpec((1, HIDDEN, D_MODEL), lambda i, m: (m[0, i], 0, 0)),
        ],
        out_specs=pl.BlockSpec((TM, D_MODEL), lambda i, m: (i, 0)),
    )
    return pl.pallas_call(
        _grouped_swiglu_kernel,
        grid_spec=grid_spec,
        out_shape=jax.ShapeDtypeStruct((np_rows, D_MODEL), jnp.float32),
        compiler_params=pltpu.CompilerParams(
            dimension_semantics=("arbitrary",)),
        interpret=_INTERPRET,
    )(meta, xs, Wg, Wu, Wd)


def _shared_swiglu_kernel(x_ref, wg_ref, wu_ref, wd_ref, o_ref):
    xb = x_ref[...].astype(jnp.bfloat16)
    a = jnp.dot(xb, wg_ref[...], preferred_element_type=jnp.float32)
    b = jnp.dot(xb, wu_ref[...], preferred_element_type=jnp.float32)
    g = ((a * jax.nn.sigmoid(a)) * b).astype(jnp.bfloat16)
    o_ref[...] = jnp.dot(g, wd_ref[...], preferred_element_type=jnp.float32)


def _shared_swiglu(x_flat, sWg, sWu, sWd):
    t = x_flat.shape[0]
    return pl.pallas_call(
        _shared_swiglu_kernel,
        grid=(t // TM,),
        in_specs=[
            pl.BlockSpec((TM, D_MODEL), lambda i: (i, 0)),
            pl.BlockSpec((D_MODEL, HIDDEN), lambda i: (0, 0)),
            pl.BlockSpec((D_MODEL, HIDDEN), lambda i: (0, 0)),
            pl.BlockSpec((HIDDEN, D_MODEL), lambda i: (0, 0)),
        ],
        out_specs=pl.BlockSpec((TM, D_MODEL), lambda i: (i, 0)),
        out_shape=jax.ShapeDtypeStruct((t, D_MODEL), jnp.float32),
        compiler_params=pltpu.CompilerParams(
            dimension_semantics=("arbitrary",)),
        interpret=_INTERPRET,
    )(x_flat, sWg, sWu, sWd)


def _combine_kernel(sh_ref, b0_ref, b1_ref, g0_ref, g1_ref, o_ref):
    o_ref[...] = (sh_ref[...]
                  + g0_ref[:, :1] * b0_ref[...]
                  + g1_ref[:, :1] * b1_ref[...])


def _combine(shared, buf, g0, g1):
    t = shared.shape[0]
    nb = t // TM
    return pl.pallas_call(
        _combine_kernel,
        grid=(nb,),
        in_specs=[
            pl.BlockSpec((TM, D_MODEL), lambda i: (i, 0)),
            pl.BlockSpec((TM, D_MODEL), lambda i: (i, 0)),
            pl.BlockSpec((TM, D_MODEL), lambda i, nb=nb: (i + nb, 0)),
            pl.BlockSpec((TM, 128), lambda i: (i, 0)),
            pl.BlockSpec((TM, 128), lambda i: (i, 0)),
        ],
        out_specs=pl.BlockSpec((TM, D_MODEL), lambda i: (i, 0)),
        out_shape=jax.ShapeDtypeStruct((t, D_MODEL), jnp.float32),
        compiler_params=pltpu.CompilerParams(
            dimension_semantics=("arbitrary",)),
        interpret=_INTERPRET,
    )(shared, buf, buf, g0, g1)


def kernel(x, Wr, Wg, Wu, Wd, sWg, sWu, sWd):
    b, s, d = x.shape
    t = b * s
    a_total = t * TOP_K
    nt = a_total // TM + N_EXPERTS
    np_rows = nt * TM
    x_flat = x.reshape(t, d)
    Wg = Wg.astype(jnp.bfloat16)
    Wu = Wu.astype(jnp.bfloat16)
    Wd = Wd.astype(jnp.bfloat16)
    sWg = sWg.astype(jnp.bfloat16)
    sWu = sWu.astype(jnp.bfloat16)
    sWd = sWd.astype(jnp.bfloat16)

    logits = x_flat @ Wr
    idx1 = jnp.argmax(logits, axis=-1)
    l1 = jnp.max(logits, axis=-1)
    masked = jnp.where(jnp.arange(N_EXPERTS)[None, :] == idx1[:, None],
                       -jnp.inf, logits)
    idx2 = jnp.argmax(masked, axis=-1)
    l2 = jnp.max(masked, axis=-1)
    m = jnp.maximum(l1, l2)
    e1 = jnp.exp(l1 - m)
    e2 = jnp.exp(l2 - m)
    zs = e1 + e2
    gate = jnp.stack([e1 / zs, e2 / zs], axis=-1)
    top_idx = jnp.stack([idx1, idx2], axis=-1).astype(jnp.int32)

    e_flat = jnp.concatenate([top_idx[:, 0], top_idx[:, 1]])
    onehot = (e_flat[:, None] == jnp.arange(N_EXPERTS)[None, :]).astype(jnp.int32)
    counts = onehot.sum(axis=0)
    nt_e = (counts + TM - 1) // TM
    cum_nt = jnp.cumsum(nt_e)
    first_tile = cum_nt - nt_e
    nt_total = cum_nt[-1]
    group_start = first_tile * TM
    rank = jnp.sum((jnp.cumsum(onehot, axis=0) - onehot) * onehot, axis=1)
    pos = group_start[e_flat] + rank
    tok = jnp.concatenate([jnp.arange(t, dtype=jnp.int32)] * 2)
    sorted_tok = jnp.zeros((np_rows,), jnp.int32).at[pos].set(
        tok, mode='promise_in_bounds', unique_indices=True)

    ti = jnp.arange(nt, dtype=jnp.int32)
    tile_e = jnp.clip(jnp.searchsorted(cum_nt, ti, side='right'), 0,
                      N_EXPERTS - 1).astype(jnp.int32)
    rem = counts[tile_e] - (ti - first_tile[tile_e]) * TM
    rem = jnp.where(ti < nt_total, rem, 0)
    meta = jnp.stack([tile_e, rem]).astype(jnp.int32)

    xs = jnp.take(x_flat, sorted_tok, axis=0)

    shared = _shared_swiglu(x_flat, sWg, sWu, sWd)

    y = _grouped_swiglu(meta, xs, Wg, Wu, Wd, nt)

    buf = jnp.take(y, pos, axis=0)
    g0 = jnp.broadcast_to(gate[:, 0:1], (t, 128))
    g1 = jnp.broadcast_to(gate[:, 1:2], (t, 128))

    out = _combine(shared, buf, g0, g1)
    return out.reshape(b, s, d)


# speedup vs baseline: 2.7786x; 1.2442x over previous
import functools

import jax
import jax.numpy as jnp
from jax import lax
from jax.experimental import pallas as pl
from jax.experimental.pallas import tpu as pltpu
from jax.experimental.pallas import tpu_sc as plsc

D_MODEL = 1024
HIDDEN = 2048
N_EXPERTS = 8
TOP_K = 2

TM = 512

_INTERPRET = False


_SC_CHUNK = 32


def _sc_dispatch(x_flat, pos3, np_rows, nw, nch):
    t, d = x_flat.shape
    mesh = plsc.VectorSubcoreMesh(core_axis_name="c", subcore_axis_name="s")

    @functools.partial(
        pl.kernel, mesh=mesh,
        out_type=jax.ShapeDtypeStruct((np_rows, d), jnp.float32),
        scratch_types=[
            pltpu.VMEM((nch, _SC_CHUNK), jnp.int32),
            pltpu.VMEM((_SC_CHUNK, d), jnp.float32),
            pltpu.SemaphoreType.DMA,
        ],
    )
    def k(x_hbm, pos_hbm, xs_hbm, idx_v, rows_v, sem):
        nc = 2
        wid = lax.axis_index("s") * nc + lax.axis_index("c")
        base = wid * (nch * _SC_CHUNK)
        tokbase = lax.rem(base, t)
        pltpu.sync_copy(pos_hbm.at[wid], idx_v)

        def body(ci, _):
            pltpu.sync_copy(
                x_hbm.at[pl.ds(tokbase + ci * _SC_CHUNK, _SC_CHUNK)], rows_v)
            pltpu.async_copy(rows_v, xs_hbm.at[idx_v.at[ci]], sem).wait()
            return 0

        lax.fori_loop(0, nch, body, 0)

    return k(x_flat, pos3)


def _grouped_swiglu_kernel(meta_ref, x_ref, wg_ref, wu_ref, wd_ref, o_ref):
    i = pl.program_id(0)
    xb = x_ref[...].astype(jnp.bfloat16)
    a = jnp.dot(xb, wg_ref[0], preferred_element_type=jnp.float32)
    b = jnp.dot(xb, wu_ref[0], preferred_element_type=jnp.float32)
    g = ((a * jax.nn.sigmoid(a)) * b).astype(jnp.bfloat16)
    contrib = jnp.dot(g, wd_ref[0], preferred_element_type=jnp.float32)
    rem = meta_ref[1, i]
    rows = lax.broadcasted_iota(jnp.int32, (TM, 1), 0)
    o_ref[...] = jnp.where(rows < rem, contrib, 0.0)


def _grouped_swiglu(meta, xs, Wg, Wu, Wd, nt):
    np_rows = nt * TM
    grid_spec = pltpu.PrefetchScalarGridSpec(
        num_scalar_prefetch=1,
        grid=(nt,),
        in_specs=[
            pl.BlockSpec((TM, D_MODEL), lambda i, m: (i, 0)),
            pl.BlockSpec((1, D_MODEL, HIDDEN), lambda i, m: (m[0, i], 0, 0)),
            pl.BlockSpec((1, D_MODEL, HIDDEN), lambda i, m: (m[0, i], 0, 0)),
            pl.BlockSpec((1, HIDDEN, D_MODEL), lambda i, m: (m[0, i], 0, 0)),
        ],
        out_specs=pl.BlockSpec((TM, D_MODEL), lambda i, m: (i, 0)),
    )
    return pl.pallas_call(
        _grouped_swiglu_kernel,
        grid_spec=grid_spec,
        out_shape=jax.ShapeDtypeStruct((np_rows, D_MODEL), jnp.float32),
        compiler_params=pltpu.CompilerParams(
            dimension_semantics=("arbitrary",)),
        interpret=_INTERPRET,
    )(meta, xs, Wg, Wu, Wd)


def _shared_swiglu_kernel(x_ref, wg_ref, wu_ref, wd_ref, o_ref):
    xb = x_ref[...].astype(jnp.bfloat16)
    a = jnp.dot(xb, wg_ref[...], preferred_element_type=jnp.float32)
    b = jnp.dot(xb, wu_ref[...], preferred_element_type=jnp.float32)
    g = ((a * jax.nn.sigmoid(a)) * b).astype(jnp.bfloat16)
    o_ref[...] = jnp.dot(g, wd_ref[...], preferred_element_type=jnp.float32)


def _shared_swiglu(x_flat, sWg, sWu, sWd):
    t = x_flat.shape[0]
    return pl.pallas_call(
        _shared_swiglu_kernel,
        grid=(t // TM,),
        in_specs=[
            pl.BlockSpec((TM, D_MODEL), lambda i: (i, 0)),
            pl.BlockSpec((D_MODEL, HIDDEN), lambda i: (0, 0)),
            pl.BlockSpec((D_MODEL, HIDDEN), lambda i: (0, 0)),
            pl.BlockSpec((HIDDEN, D_MODEL), lambda i: (0, 0)),
        ],
        out_specs=pl.BlockSpec((TM, D_MODEL), lambda i: (i, 0)),
        out_shape=jax.ShapeDtypeStruct((t, D_MODEL), jnp.float32),
        compiler_params=pltpu.CompilerParams(
            dimension_semantics=("arbitrary",)),
        interpret=_INTERPRET,
    )(x_flat, sWg, sWu, sWd)


def _combine_kernel(sh_ref, b0_ref, b1_ref, g0_ref, g1_ref, o_ref):
    o_ref[...] = (sh_ref[...]
                  + g0_ref[:, :1] * b0_ref[...]
                  + g1_ref[:, :1] * b1_ref[...])


def _combine(shared, buf, g0, g1):
    t = shared.shape[0]
    nb = t // TM
    return pl.pallas_call(
        _combine_kernel,
        grid=(nb,),
        in_specs=[
            pl.BlockSpec((TM, D_MODEL), lambda i: (i, 0)),
            pl.BlockSpec((TM, D_MODEL), lambda i: (i, 0)),
            pl.BlockSpec((TM, D_MODEL), lambda i, nb=nb: (i + nb, 0)),
            pl.BlockSpec((TM, 128), lambda i: (i, 0)),
            pl.BlockSpec((TM, 128), lambda i: (i, 0)),
        ],
        out_specs=pl.BlockSpec((TM, D_MODEL), lambda i: (i, 0)),
        out_shape=jax.ShapeDtypeStruct((t, D_MODEL), jnp.float32),
        compiler_params=pltpu.CompilerParams(
            dimension_semantics=("arbitrary",)),
        interpret=_INTERPRET,
    )(shared, buf, buf, g0, g1)


def kernel(x, Wr, Wg, Wu, Wd, sWg, sWu, sWd):
    b, s, d = x.shape
    t = b * s
    a_total = t * TOP_K
    nt = a_total // TM + N_EXPERTS
    np_rows = nt * TM
    x_flat = x.reshape(t, d)
    Wg = Wg.astype(jnp.bfloat16)
    Wu = Wu.astype(jnp.bfloat16)
    Wd = Wd.astype(jnp.bfloat16)
    sWg = sWg.astype(jnp.bfloat16)
    sWu = sWu.astype(jnp.bfloat16)
    sWd = sWd.astype(jnp.bfloat16)

    logits = x_flat @ Wr
    idx1 = jnp.argmax(logits, axis=-1)
    l1 = jnp.max(logits, axis=-1)
    masked = jnp.where(jnp.arange(N_EXPERTS)[None, :] == idx1[:, None],
                       -jnp.inf, logits)
    idx2 = jnp.argmax(masked, axis=-1)
    l2 = jnp.max(masked, axis=-1)
    m = jnp.maximum(l1, l2)
    e1 = jnp.exp(l1 - m)
    e2 = jnp.exp(l2 - m)
    zs = e1 + e2
    gate = jnp.stack([e1 / zs, e2 / zs], axis=-1)
    top_idx = jnp.stack([idx1, idx2], axis=-1).astype(jnp.int32)

    e_flat = jnp.concatenate([top_idx[:, 0], top_idx[:, 1]])
    onehot = (e_flat[:, None] == jnp.arange(N_EXPERTS)[None, :]).astype(jnp.int32)
    counts = onehot.sum(axis=0)
    nt_e = (counts + TM - 1) // TM
    cum_nt = jnp.cumsum(nt_e)
    first_tile = cum_nt - nt_e
    nt_total = cum_nt[-1]
    group_start = first_tile * TM
    rank = jnp.sum((jnp.cumsum(onehot, axis=0) - onehot) * onehot, axis=1)
    pos = group_start[e_flat] + rank
    ti = jnp.arange(nt, dtype=jnp.int32)
    tile_e = jnp.clip(jnp.searchsorted(cum_nt, ti, side='right'), 0,
                      N_EXPERTS - 1).astype(jnp.int32)
    rem = counts[tile_e] - (ti - first_tile[tile_e]) * TM
    rem = jnp.where(ti < nt_total, rem, 0)
    meta = jnp.stack([tile_e, rem]).astype(jnp.int32)

    nw = 32
    nch = a_total // (nw * _SC_CHUNK)
    pos3 = pos.reshape(nw, nch, _SC_CHUNK)
    xs = _sc_dispatch(x_flat, pos3, np_rows, nw, nch)

    shared = _shared_swiglu(x_flat, sWg, sWu, sWd)

    y = _grouped_swiglu(meta, xs, Wg, Wu, Wd, nt)

    buf = jnp.take(y, pos, axis=0)
    g0 = jnp.broadcast_to(gate[:, 0:1], (t, 128))
    g1 = jnp.broadcast_to(gate[:, 1:2], (t, 128))

    out = _combine(shared, buf, g0, g1)
    return out.reshape(b, s, d)


# speedup vs baseline: 3.0090x; 1.0829x over previous
import functools

import jax
import jax.numpy as jnp
from jax import lax
from jax.experimental import pallas as pl
from jax.experimental.pallas import tpu as pltpu
from jax.experimental.pallas import tpu_sc as plsc

D_MODEL = 1024
HIDDEN = 2048
N_EXPERTS = 8
TOP_K = 2

TM = 512

_INTERPRET = False


_SC_CHUNK = 32


def _sc_dispatch(x_flat, pos3, np_rows, nw, nch):
    t, d = x_flat.shape
    mesh = plsc.VectorSubcoreMesh(core_axis_name="c", subcore_axis_name="s")

    @functools.partial(
        pl.kernel, mesh=mesh,
        out_type=jax.ShapeDtypeStruct((np_rows, d), jnp.float32),
        scratch_types=[
            pltpu.VMEM((nch, _SC_CHUNK), jnp.int32),
            pltpu.VMEM((_SC_CHUNK, d), jnp.float32),
            pltpu.SemaphoreType.DMA,
        ],
    )
    def k(x_hbm, pos_hbm, xs_hbm, idx_v, rows_v, sem):
        nc = 2
        wid = lax.axis_index("s") * nc + lax.axis_index("c")
        base = wid * (nch * _SC_CHUNK)
        tokbase = lax.rem(base, t)
        pltpu.sync_copy(pos_hbm.at[wid], idx_v)

        def body(ci, _):
            pltpu.sync_copy(
                x_hbm.at[pl.ds(tokbase + ci * _SC_CHUNK, _SC_CHUNK)], rows_v)
            pltpu.async_copy(rows_v, xs_hbm.at[idx_v.at[ci]], sem).wait()
            return 0

        lax.fori_loop(0, nch, body, 0)

    return k(x_flat, pos3)


def _sc_collect(y, pos3, a_total, nch):
    d = y.shape[1]
    mesh = plsc.VectorSubcoreMesh(core_axis_name="c", subcore_axis_name="s")

    @functools.partial(
        pl.kernel, mesh=mesh,
        out_type=jax.ShapeDtypeStruct((a_total, d), jnp.float32),
        scratch_types=[
            pltpu.VMEM((nch, _SC_CHUNK), jnp.int32),
            pltpu.VMEM((_SC_CHUNK, d), jnp.float32),
            pltpu.SemaphoreType.DMA,
        ],
    )
    def k(y_hbm, pos_hbm, buf_hbm, idx_v, rows_v, sem):
        nc = 2
        wid = lax.axis_index("s") * nc + lax.axis_index("c")
        base = wid * (nch * _SC_CHUNK)
        pltpu.sync_copy(pos_hbm.at[wid], idx_v)

        def body(ci, _):
            pltpu.async_copy(y_hbm.at[idx_v.at[ci]], rows_v, sem).wait()
            pltpu.sync_copy(
                rows_v, buf_hbm.at[pl.ds(base + ci * _SC_CHUNK, _SC_CHUNK)])
            return 0

        lax.fori_loop(0, nch, body, 0)

    return k(y, pos3)


def _grouped_swiglu_kernel(meta_ref, x_ref, wg_ref, wu_ref, wd_ref, o_ref):
    i = pl.program_id(0)
    xb = x_ref[...].astype(jnp.bfloat16)
    a = jnp.dot(xb, wg_ref[0], preferred_element_type=jnp.float32)
    b = jnp.dot(xb, wu_ref[0], preferred_element_type=jnp.float32)
    g = ((a * jax.nn.sigmoid(a)) * b).astype(jnp.bfloat16)
    contrib = jnp.dot(g, wd_ref[0], preferred_element_type=jnp.float32)
    rem = meta_ref[1, i]
    rows = lax.broadcasted_iota(jnp.int32, (TM, 1), 0)
    o_ref[...] = jnp.where(rows < rem, contrib, 0.0)


def _grouped_swiglu(meta, xs, Wg, Wu, Wd, nt):
    np_rows = nt * TM
    grid_spec = pltpu.PrefetchScalarGridSpec(
        num_scalar_prefetch=1,
        grid=(nt,),
        in_specs=[
            pl.BlockSpec((TM, D_MODEL), lambda i, m: (i, 0)),
            pl.BlockSpec((1, D_MODEL, HIDDEN), lambda i, m: (m[0, i], 0, 0)),
            pl.BlockSpec((1, D_MODEL, HIDDEN), lambda i, m: (m[0, i], 0, 0)),
            pl.BlockSpec((1, HIDDEN, D_MODEL), lambda i, m: (m[0, i], 0, 0)),
        ],
        out_specs=pl.BlockSpec((TM, D_MODEL), lambda i, m: (i, 0)),
    )
    return pl.pallas_call(
        _grouped_swiglu_kernel,
        grid_spec=grid_spec,
        out_shape=jax.ShapeDtypeStruct((np_rows, D_MODEL), jnp.float32),
        compiler_params=pltpu.CompilerParams(
            dimension_semantics=("arbitrary",)),
        interpret=_INTERPRET,
    )(meta, xs, Wg, Wu, Wd)


def _shared_swiglu_kernel(x_ref, wg_ref, wu_ref, wd_ref, o_ref):
    xb = x_ref[...].astype(jnp.bfloat16)
    a = jnp.dot(xb, wg_ref[...], preferred_element_type=jnp.float32)
    b = jnp.dot(xb, wu_ref[...], preferred_element_type=jnp.float32)
    g = ((a * jax.nn.sigmoid(a)) * b).astype(jnp.bfloat16)
    o_ref[...] = jnp.dot(g, wd_ref[...], preferred_element_type=jnp.float32)


def _shared_swiglu(x_flat, sWg, sWu, sWd):
    t = x_flat.shape[0]
    return pl.pallas_call(
        _shared_swiglu_kernel,
        grid=(t // TM,),
        in_specs=[
            pl.BlockSpec((TM, D_MODEL), lambda i: (i, 0)),
            pl.BlockSpec((D_MODEL, HIDDEN), lambda i: (0, 0)),
            pl.BlockSpec((D_MODEL, HIDDEN), lambda i: (0, 0)),
            pl.BlockSpec((HIDDEN, D_MODEL), lambda i: (0, 0)),
        ],
        out_specs=pl.BlockSpec((TM, D_MODEL), lambda i: (i, 0)),
        out_shape=jax.ShapeDtypeStruct((t, D_MODEL), jnp.float32),
        compiler_params=pltpu.CompilerParams(
            dimension_semantics=("arbitrary",)),
        interpret=_INTERPRET,
    )(x_flat, sWg, sWu, sWd)


def _combine_kernel(sh_ref, b0_ref, b1_ref, g0_ref, g1_ref, o_ref):
    o_ref[...] = (sh_ref[...]
                  + g0_ref[:, :1] * b0_ref[...]
                  + g1_ref[:, :1] * b1_ref[...])


def _combine(shared, buf, g0, g1):
    t = shared.shape[0]
    nb = t // TM
    return pl.pallas_call(
        _combine_kernel,
        grid=(nb,),
        in_specs=[
            pl.BlockSpec((TM, D_MODEL), lambda i: (i, 0)),
            pl.BlockSpec((TM, D_MODEL), lambda i: (i, 0)),
            pl.BlockSpec((TM, D_MODEL), lambda i, nb=nb: (i + nb, 0)),
            pl.BlockSpec((TM, 128), lambda i: (i, 0)),
            pl.BlockSpec((TM, 128), lambda i: (i, 0)),
        ],
        out_specs=pl.BlockSpec((TM, D_MODEL), lambda i: (i, 0)),
        out_shape=jax.ShapeDtypeStruct((t, D_MODEL), jnp.float32),
        compiler_params=pltpu.CompilerParams(
            dimension_semantics=("arbitrary",)),
        interpret=_INTERPRET,
    )(shared, buf, buf, g0, g1)


def kernel(x, Wr, Wg, Wu, Wd, sWg, sWu, sWd):
    b, s, d = x.shape
    t = b * s
    a_total = t * TOP_K
    nt = a_total // TM + N_EXPERTS
    np_rows = nt * TM
    x_flat = x.reshape(t, d)
    Wg = Wg.astype(jnp.bfloat16)
    Wu = Wu.astype(jnp.bfloat16)
    Wd = Wd.astype(jnp.bfloat16)
    sWg = sWg.astype(jnp.bfloat16)
    sWu = sWu.astype(jnp.bfloat16)
    sWd = sWd.astype(jnp.bfloat16)

    logits = x_flat @ Wr
    idx1 = jnp.argmax(logits, axis=-1)
    l1 = jnp.max(logits, axis=-1)
    masked = jnp.where(jnp.arange(N_EXPERTS)[None, :] == idx1[:, None],
                       -jnp.inf, logits)
    idx2 = jnp.argmax(masked, axis=-1)
    l2 = jnp.max(masked, axis=-1)
    m = jnp.maximum(l1, l2)
    e1 = jnp.exp(l1 - m)
    e2 = jnp.exp(l2 - m)
    zs = e1 + e2
    gate = jnp.stack([e1 / zs, e2 / zs], axis=-1)
    top_idx = jnp.stack([idx1, idx2], axis=-1).astype(jnp.int32)

    e_flat = jnp.concatenate([top_idx[:, 0], top_idx[:, 1]])
    onehot = (e_flat[:, None] == jnp.arange(N_EXPERTS)[None, :]).astype(jnp.int32)
    counts = onehot.sum(axis=0)
    nt_e = (counts + TM - 1) // TM
    cum_nt = jnp.cumsum(nt_e)
    first_tile = cum_nt - nt_e
    nt_total = cum_nt[-1]
    group_start = first_tile * TM
    rank = jnp.sum((jnp.cumsum(onehot, axis=0) - onehot) * onehot, axis=1)
    pos = group_start[e_flat] + rank
    ti = jnp.arange(nt, dtype=jnp.int32)
    tile_e = jnp.clip(jnp.searchsorted(cum_nt, ti, side='right'), 0,
                      N_EXPERTS - 1).astype(jnp.int32)
    rem = counts[tile_e] - (ti - first_tile[tile_e]) * TM
    rem = jnp.where(ti < nt_total, rem, 0)
    meta = jnp.stack([tile_e, rem]).astype(jnp.int32)

    nw = 32
    nch = a_total // (nw * _SC_CHUNK)
    pos3 = pos.reshape(nw, nch, _SC_CHUNK)
    xs = _sc_dispatch(x_flat, pos3, np_rows, nw, nch)

    shared = _shared_swiglu(x_flat, sWg, sWu, sWd)

    y = _grouped_swiglu(meta, xs, Wg, Wu, Wd, nt)

    buf = _sc_collect(y, pos3, a_total, nch)
    g0 = jnp.broadcast_to(gate[:, 0:1], (t, 128))
    g1 = jnp.broadcast_to(gate[:, 1:2], (t, 128))

    out = _combine(shared, buf, g0, g1)
    return out.reshape(b, s, d)
